# baseline (speedup 1.0000x reference)
import jax
import jax.numpy as jnp
from jax import lax
from jax.experimental import pallas as pl
from jax.experimental.pallas import tpu as pltpu
from jax.experimental.pallas import tpu_sc as plsc

D = 64
DP = 128
NC, NS = 2, 16
NW = NC * NS
B = 4096 * 200
C = 128
BPW = B // NW
G = BPW // C
NBUF = 4


def _emb_body(idx_hbm, tab_hbm, out_hbm, idx_v,
              b0, b1, b2, b3, g0, g1, g2, g3, s0, s1, s2, s3):
    w = lax.axis_index("s") * NC + lax.axis_index("c")
    base = w * BPW
    pltpu.sync_copy(idx_hbm.at[pl.ds(w * G, G)], idx_v)

    buf = (b0, b1, b2, b3)
    gsem = (g0, g1, g2, g3)
    ssem = (s0, s1, s2, s3)

    for b in range(NBUF):
        pltpu.async_copy(tab_hbm.at[idx_v.at[b]], buf[b], gsem[b])

    def chunk(gc, b, start_next):
        pltpu.make_async_copy(tab_hbm.at[idx_v.at[gc]], buf[b], gsem[b]).wait()
        dst = out_hbm.at[pl.ds(base + gc * C, C)]
        pltpu.async_copy(buf[b], dst, ssem[b])
        pltpu.make_async_copy(buf[b], dst, ssem[b]).wait()
        if start_next:
            pltpu.async_copy(tab_hbm.at[idx_v.at[gc + NBUF]], buf[b], gsem[b])

    def step(i, carry):
        g = NBUF * i
        for b in range(NBUF):
            chunk(g + b, b, True)
        return carry

    lax.fori_loop(0, G // NBUF - 1, step, 0, unroll=False)
    for b in range(NBUF):
        chunk(G - NBUF + b, b, False)


def kernel(x, embed_weight):
    s0, s1 = x.shape
    xf = x.reshape(-1).astype(jnp.int32)
    p = (xf >> 1).reshape(NW * G, C)
    tabp = embed_weight.reshape(embed_weight.shape[0] // 2, DP)
    mesh = plsc.VectorSubcoreMesh(
        core_axis_name="c", subcore_axis_name="s",
        num_cores=NC, num_subcores=NS,
    )
    k = pl.kernel(
        _emb_body,
        out_type=jax.ShapeDtypeStruct((B, DP), jnp.float32),
        mesh=mesh,
        scratch_types=[
            pltpu.VMEM((G, C), jnp.int32),
        ] + [pltpu.VMEM((C, DP), jnp.float32)] * NBUF
          + [pltpu.SemaphoreType.DMA] * (2 * NBUF),
    )
    pairs = k(p, tabp)
    odd = (xf & 1).astype(jnp.bool_)
    out = jnp.where(odd[:, None], pairs[:, D:], pairs[:, :D])
    return out.reshape(s0, s1, D)


# speedup vs baseline: 1.2362x; 1.2362x over previous
import jax
import jax.numpy as jnp
from jax import lax
from jax.experimental import pallas as pl
from jax.experimental.pallas import tpu as pltpu
from jax.experimental.pallas import tpu_sc as plsc

D = 64
DP = 128
NC, NS = 2, 16
NW = NC * NS
B = 4096 * 200
C = 128
BPW = B // NW
G = BPW // C
NBUF = 2


def _emb_body(idx_hbm, par_hbm, tab_hbm, out_hbm, idx_v, par_v,
              b0, b1, c0, g0, g1, s0, s1):
    w = lax.axis_index("s") * NC + lax.axis_index("c")
    base = w * BPW
    pltpu.sync_copy(idx_hbm.at[pl.ds(w * G, G)], idx_v)
    pltpu.sync_copy(par_hbm.at[pl.ds(w * G, G)], par_v)

    buf = (b0, b1)
    cbuf = (c0,)
    gsem = (g0, g1)
    ssem = (s0, s1)

    for b in range(NBUF):
        pltpu.async_copy(tab_hbm.at[idx_v.at[b]], buf[b], gsem[b])

    def chunk(gc, b, start_next):
        pltpu.make_async_copy(tab_hbm.at[idx_v.at[gc]], buf[b], gsem[b]).wait()

        cb = cbuf[0]

        def rowgrp(rg, carry):
            r0 = rg * 16
            pvec = par_v[gc, pl.ds(r0, 16)]
            for k in range(16):
                off = pvec[k]
                r = r0 + k
                for q in range(D // 16):
                    cb[r, pl.ds(q * 16, 16)] = (
                        buf[b][r, pl.ds(off + q * 16, 16)])
            return carry

        lax.fori_loop(0, C // 16, rowgrp, 0, unroll=False)
        dst = out_hbm.at[pl.ds(base + gc * C, C)]
        pltpu.async_copy(cb, dst, ssem[b])
        pltpu.make_async_copy(cb, dst, ssem[b]).wait()
        if start_next:
            pltpu.async_copy(tab_hbm.at[idx_v.at[gc + NBUF]], buf[b], gsem[b])

    def step(i, carry):
        g = NBUF * i
        for b in range(NBUF):
            chunk(g + b, b, True)
        return carry

    lax.fori_loop(0, G // NBUF - 1, step, 0, unroll=False)
    for b in range(NBUF):
        chunk(G - NBUF + b, b, False)


def kernel(x, embed_weight):
    s0, s1 = x.shape
    xf = x.reshape(-1).astype(jnp.int32)
    p = (xf >> 1).reshape(NW * G, C)
    par = ((xf & 1) * D).reshape(NW * G, C)
    tabp = embed_weight.reshape(embed_weight.shape[0] // 2, DP)
    mesh = plsc.VectorSubcoreMesh(
        core_axis_name="c", subcore_axis_name="s",
        num_cores=NC, num_subcores=NS,
    )
    k = pl.kernel(
        _emb_body,
        out_type=jax.ShapeDtypeStruct((B, D), jnp.float32),
        mesh=mesh,
        scratch_types=[
            pltpu.VMEM((G, C), jnp.int32),
            pltpu.VMEM((G, C), jnp.int32),
        ] + [pltpu.VMEM((C, DP), jnp.float32)] * NBUF
          + [pltpu.VMEM((C, D), jnp.float32)] * 1
          + [pltpu.SemaphoreType.DMA] * (2 * NBUF),
    )
    out = k(p, par, tabp)
    return out.reshape(s0, s1, D)
